# baseline (speedup 1.0000x reference)
import functools

import jax
import jax.numpy as jnp
from jax import lax
from jax.experimental import pallas as pl
from jax.experimental.pallas import tpu as pltpu
from jax.experimental.pallas import tpu_sc as plsc

_NC = 2
_NS = 16
_NW = _NC * _NS

_CHUNK = 256
_NBUF = 4

_RB = 384


def _mesh():
    return plsc.VectorSubcoreMesh(
        core_axis_name="c", subcore_axis_name="s",
        num_cores=_NC, num_subcores=_NS)


def _wid():
    return lax.axis_index("s") * _NC + lax.axis_index("c")


def _build_relayout(V, C):
    nfull = V // _RB
    tail = V - nfull * _RB
    assert C == 64 and _RB % 128 == 0 and tail % 8 == 0

    @functools.partial(
        pl.kernel,
        out_type=jax.ShapeDtypeStruct((V // 2, 2 * C), jnp.float32),
        mesh=_mesh(),
        scratch_types=[
            [pltpu.VMEM((C, _RB), jnp.float32)] * 2,
            [pltpu.VMEM((_RB // 2, 2 * C), jnp.float32)] * 2,
            pltpu.VMEM((tail // 2, 2 * C), jnp.float32),
            [pltpu.SemaphoreType.DMA] * 2,
            [pltpu.SemaphoreType.DMA] * 2,
        ],
        compiler_params=pltpu.CompilerParams(
            use_tc_tiling_on_sc=True, needs_layout_passes=False),
    )
    def t(tt_hbm, tail_hbm, pack_hbm, bins, bouts, tbuf, isems, osems):
        w = _wid()
        nw = (nfull - w + _NW - 1) // _NW

        lanes = lax.iota(jnp.int32, 16)

        def load_start(t_, b):
            blk = w + t_ * _NW
            pltpu.async_copy(
                tt_hbm.at[:, pl.ds(blk * _RB, _RB)], bins[b], isems[b])

        def load_wait(b):
            pltpu.make_async_copy(
                tt_hbm.at[:, pl.ds(0, _RB)], bins[b], isems[b]).wait()

        def store_start(t_, b):
            blk = w + t_ * _NW
            pltpu.async_copy(
                bouts[b],
                pack_hbm.at[pl.ds(blk * (_RB // 2), _RB // 2)], osems[b])

        def store_wait(b):
            pltpu.make_async_copy(
                bouts[b], pack_hbm.at[pl.ds(0, _RB // 2)], osems[b]).wait()

        cks = [lanes + 16 * k for k in range(C // 16)]

        def transpose(bin_ref, bout_ref, ncols):
            @plsc.parallel_loop(0, ncols // 2, unroll=16)
            def _row(q):
                for h in range(2):
                    jj = jnp.full((16,), q * 2 + h, jnp.int32)
                    for k in range(C // 16):
                        v = plsc.load_gather(bin_ref, [cks[k], jj])
                        bout_ref[q, pl.ds(h * C + 16 * k, 16)] = v

        def step(t_, b):
            @pl.when(t_ + 1 < nw)
            def _():
                load_start(t_ + 1, 1 - b)

            load_wait(b)

            @pl.when(t_ >= 2)
            def _():
                store_wait(b)

            transpose(bins[b], bouts[b], _RB)
            store_start(t_, b)

        @pl.when(nw > 0)
        def _():
            load_start(0, 0)

            def body2(p, carry):
                t0 = p * 2
                step(t0, 0)

                @pl.when(t0 + 1 < nw)
                def _():
                    step(t0 + 1, 1)

                return carry

            lax.fori_loop(0, (nw + 1) // 2, body2, 0)

            par = lax.rem(nw, 2)

            @pl.when(jnp.logical_and(nw >= 2, par == 0))
            def _():
                store_wait(0)

            @pl.when(jnp.logical_and(nw >= 2, par == 1))
            def _():
                store_wait(1)

            @pl.when(par == 0)
            def _():
                store_wait(1)

            @pl.when(par == 1)
            def _():
                store_wait(0)

        if tail:
            @pl.when(w == 0)
            def _():
                pltpu.sync_copy(tail_hbm, tbuf)
                pltpu.sync_copy(
                    tbuf, pack_hbm.at[pl.ds(nfull * (_RB // 2), tail // 2)])

    return t


def _build_gather(B, C, b_per_w, ch):
    nch = b_per_w // ch
    assert nch % _NBUF == 0 and nch >= 2 * _NBUF

    @functools.partial(
        pl.kernel,
        out_type=jax.ShapeDtypeStruct((B, 2, C), jnp.float32),
        mesh=_mesh(),
        scratch_types=[
            pltpu.VMEM((b_per_w,), jnp.int32),
            [pltpu.VMEM((ch, C), jnp.float32)] * _NBUF,
            [pltpu.SemaphoreType.DMA] * _NBUF,
            [pltpu.SemaphoreType.DMA] * _NBUF,
        ],
        compiler_params=pltpu.CompilerParams(use_tc_tiling_on_sc=False),
    )
    def k(idx_hbm, table_hbm, out_hbm, idx_v, rows, gsem, wsem):
        base = _wid() * b_per_w
        pltpu.sync_copy(idx_hbm.at[pl.ds(base, b_per_w)], idx_v)

        def gather_start(g, b):
            pltpu.async_copy(
                table_hbm.at[idx_v.at[pl.ds(g * ch, ch)]], rows[b], gsem[b])

        def gather_wait(b):
            pltpu.make_async_copy(
                table_hbm.at[pl.ds(0, ch)], rows[b], gsem[b]).wait()

        def write_start(g, b):
            pltpu.async_copy(
                rows[b],
                out_hbm.at[pl.ds(base + g * ch, ch), 0], wsem[b])

        def write_wait(b):
            pltpu.make_async_copy(
                rows[b], out_hbm.at[pl.ds(base, ch), 0], wsem[b]).wait()

        gather_start(0, 0)
        gather_start(1, 1)

        def outer(j, carry):
            for b in range(_NBUF):
                g = j * _NBUF + b

                gather_wait(b)
                write_start(g, b)

                @pl.when(g >= 2)
                def _():
                    write_wait((b + _NBUF - 2) % _NBUF)

                @pl.when(g + 2 < nch)
                def _():
                    gather_start(g + 2, (b + 2) % _NBUF)

            return carry

        lax.fori_loop(0, nch // _NBUF, outer, 0)
        write_wait((nch - 2) % _NBUF)
        write_wait((nch - 1) % _NBUF)

    return k


@jax.jit
def kernel(x, table):
    b0, b1 = x.shape
    B = b0 * b1
    V, C = table.shape
    idx = x.reshape(B).astype(jnp.int32)
    nfull = V // _RB
    tail_lin = table[nfull * _RB:].reshape((V - nfull * _RB) // 2, 2 * C)
    pack = _build_relayout(V, C)(table.T, tail_lin)
    table_lin = pack.reshape(V, C)
    out = _build_gather(B, C, B // _NW, _CHUNK)(idx, table_lin)
    return out.reshape(b0, b1, 2 * C)[:, :, :C]


# speedup vs baseline: 1.0032x; 1.0032x over previous
import functools

import jax
import jax.numpy as jnp
from jax import lax
from jax.experimental import pallas as pl
from jax.experimental.pallas import tpu as pltpu
from jax.experimental.pallas import tpu_sc as plsc

_NC = 2
_NS = 16
_NW = _NC * _NS

_CHUNK = 256
_NBUF = 4

_RB = 128


def _mesh():
    return plsc.VectorSubcoreMesh(
        core_axis_name="c", subcore_axis_name="s",
        num_cores=_NC, num_subcores=_NS)


def _wid():
    return lax.axis_index("s") * _NC + lax.axis_index("c")


def _build_relayout(V, C):
    nfull = V // _RB
    tail = V - nfull * _RB
    assert C == 64 and _RB % 128 == 0 and tail % 8 == 0

    @functools.partial(
        pl.kernel,
        out_type=jax.ShapeDtypeStruct((V // 2, 2 * C), jnp.float32),
        mesh=_mesh(),
        scratch_types=[
            [pltpu.VMEM((C, _RB), jnp.float32)] * 2,
            [pltpu.VMEM((_RB // 2, 2 * C), jnp.float32)] * 2,
            pltpu.VMEM((tail // 2, 2 * C), jnp.float32),
            [pltpu.SemaphoreType.DMA] * 2,
            [pltpu.SemaphoreType.DMA] * 2,
        ],
        compiler_params=pltpu.CompilerParams(
            use_tc_tiling_on_sc=True, needs_layout_passes=False),
    )
    def t(tt_hbm, tail_hbm, pack_hbm, bins, bouts, tbuf, isems, osems):
        w = _wid()
        nw = (nfull - w + _NW - 1) // _NW

        lanes = lax.iota(jnp.int32, 16)

        def load_start(t_, b):
            blk = w + t_ * _NW
            pltpu.async_copy(
                tt_hbm.at[:, pl.ds(blk * _RB, _RB)], bins[b], isems[b])

        def load_wait(b):
            pltpu.make_async_copy(
                tt_hbm.at[:, pl.ds(0, _RB)], bins[b], isems[b]).wait()

        def store_start(t_, b):
            blk = w + t_ * _NW
            pltpu.async_copy(
                bouts[b],
                pack_hbm.at[pl.ds(blk * (_RB // 2), _RB // 2)], osems[b])

        def store_wait(b):
            pltpu.make_async_copy(
                bouts[b], pack_hbm.at[pl.ds(0, _RB // 2)], osems[b]).wait()

        cks = [lanes + 16 * k for k in range(C // 16)]

        def transpose(bin_ref, bout_ref, ncols):
            @plsc.parallel_loop(0, ncols // 2, unroll=16)
            def _row(q):
                for h in range(2):
                    jj = jnp.full((16,), q * 2 + h, jnp.int32)
                    for k in range(C // 16):
                        v = plsc.load_gather(bin_ref, [cks[k], jj])
                        bout_ref[q, pl.ds(h * C + 16 * k, 16)] = v

        def step(t_, b):
            @pl.when(t_ + 1 < nw)
            def _():
                load_start(t_ + 1, 1 - b)

            load_wait(b)

            @pl.when(t_ >= 2)
            def _():
                store_wait(b)

            transpose(bins[b], bouts[b], _RB)
            store_start(t_, b)

        @pl.when(nw > 0)
        def _():
            load_start(0, 0)

            def body2(p, carry):
                t0 = p * 2
                step(t0, 0)

                @pl.when(t0 + 1 < nw)
                def _():
                    step(t0 + 1, 1)

                return carry

            lax.fori_loop(0, (nw + 1) // 2, body2, 0)

            par = lax.rem(nw, 2)

            @pl.when(jnp.logical_and(nw >= 2, par == 0))
            def _():
                store_wait(0)

            @pl.when(jnp.logical_and(nw >= 2, par == 1))
            def _():
                store_wait(1)

            @pl.when(par == 0)
            def _():
                store_wait(1)

            @pl.when(par == 1)
            def _():
                store_wait(0)

        if tail:
            @pl.when(w == 0)
            def _():
                pltpu.sync_copy(tail_hbm, tbuf)
                pltpu.sync_copy(
                    tbuf, pack_hbm.at[pl.ds(nfull * (_RB // 2), tail // 2)])

    return t


def _build_gather(B, C, b_per_w, ch):
    nch = b_per_w // ch
    assert nch % _NBUF == 0 and nch >= 2 * _NBUF

    @functools.partial(
        pl.kernel,
        out_type=jax.ShapeDtypeStruct((B, 2, C), jnp.float32),
        mesh=_mesh(),
        scratch_types=[
            pltpu.VMEM((b_per_w,), jnp.int32),
            [pltpu.VMEM((ch, C), jnp.float32)] * _NBUF,
            [pltpu.SemaphoreType.DMA] * _NBUF,
            [pltpu.SemaphoreType.DMA] * _NBUF,
        ],
        compiler_params=pltpu.CompilerParams(use_tc_tiling_on_sc=False),
    )
    def k(idx_hbm, table_hbm, out_hbm, idx_v, rows, gsem, wsem):
        base = _wid() * b_per_w
        pltpu.sync_copy(idx_hbm.at[pl.ds(base, b_per_w)], idx_v)

        def gather_start(g, b):
            pltpu.async_copy(
                table_hbm.at[idx_v.at[pl.ds(g * ch, ch)]], rows[b], gsem[b])

        def gather_wait(b):
            pltpu.make_async_copy(
                table_hbm.at[pl.ds(0, ch)], rows[b], gsem[b]).wait()

        def write_start(g, b):
            pltpu.async_copy(
                rows[b],
                out_hbm.at[pl.ds(base + g * ch, ch), 0], wsem[b])

        def write_wait(b):
            pltpu.make_async_copy(
                rows[b], out_hbm.at[pl.ds(base, ch), 0], wsem[b]).wait()

        gather_start(0, 0)
        gather_start(1, 1)

        def outer(j, carry):
            for b in range(_NBUF):
                g = j * _NBUF + b

                gather_wait(b)
                write_start(g, b)

                @pl.when(g >= 2)
                def _():
                    write_wait((b + _NBUF - 2) % _NBUF)

                @pl.when(g + 2 < nch)
                def _():
                    gather_start(g + 2, (b + 2) % _NBUF)

            return carry

        lax.fori_loop(0, nch // _NBUF, outer, 0)
        write_wait((nch - 2) % _NBUF)
        write_wait((nch - 1) % _NBUF)

    return k


@jax.jit
def kernel(x, table):
    b0, b1 = x.shape
    B = b0 * b1
    V, C = table.shape
    idx = x.reshape(B).astype(jnp.int32)
    nfull = V // _RB
    tail_lin = table[nfull * _RB:].reshape((V - nfull * _RB) // 2, 2 * C)
    pack = _build_relayout(V, C)(table.T, tail_lin)
    table_lin = pack.reshape(V, C)
    out = _build_gather(B, C, B // _NW, _CHUNK)(idx, table_lin)
    return out.reshape(b0, b1, 2 * C)[:, :, :C]


# speedup vs baseline: 1.2116x; 1.2077x over previous
import functools

import jax
import jax.numpy as jnp
from jax import lax
from jax.experimental import pallas as pl
from jax.experimental.pallas import tpu as pltpu
from jax.experimental.pallas import tpu_sc as plsc

_NC = 2
_NS = 16
_NW = _NC * _NS

_CHUNK = 160
_NBUF = 4


def _build(B, D, b_per_w, ch):
    nch = b_per_w // ch
    assert nch % _NBUF == 0 and nch >= 2 * _NBUF
    mesh = plsc.VectorSubcoreMesh(
        core_axis_name="c", subcore_axis_name="s",
        num_cores=_NC, num_subcores=_NS)

    @functools.partial(
        pl.kernel,
        out_type=jax.ShapeDtypeStruct((B, 2, D), jnp.float32),
        mesh=mesh,
        scratch_types=[
            pltpu.VMEM((b_per_w,), jnp.int32),
            [pltpu.VMEM((ch, 2 * D), jnp.float32)] * _NBUF,
            [pltpu.SemaphoreType.DMA] * _NBUF,
            [pltpu.SemaphoreType.DMA] * _NBUF,
        ],
        compiler_params=pltpu.CompilerParams(use_tc_tiling_on_sc=False),
    )
    def k(idx_hbm, table_hbm, out_hbm, idx_v, rows, gsem, wsem):
        wid = lax.axis_index("s") * _NC + lax.axis_index("c")
        base = wid * b_per_w
        pltpu.sync_copy(idx_hbm.at[pl.ds(base, b_per_w)], idx_v)

        def gather_start(g, b):
            pltpu.async_copy(
                table_hbm.at[idx_v.at[pl.ds(g * ch, ch)]], rows[b], gsem[b])

        def gather_wait(b):
            pltpu.make_async_copy(
                table_hbm.at[pl.ds(0, ch)], rows[b], gsem[b]).wait()

        def write_start(g, b):
            pltpu.async_copy(
                rows[b].at[:, pl.ds(0, D)],
                out_hbm.at[pl.ds(base + g * ch, ch), 0], wsem[b])

        def write_wait(b):
            pltpu.make_async_copy(
                rows[b].at[:, pl.ds(0, D)],
                out_hbm.at[pl.ds(base, ch), 0], wsem[b]).wait()

        gather_start(0, 0)
        gather_start(1, 1)

        def outer(j, carry):
            for b in range(_NBUF):
                g = j * _NBUF + b

                gather_wait(b)
                write_start(g, b)

                @pl.when(g >= 2)
                def _():
                    write_wait((b + _NBUF - 2) % _NBUF)

                @pl.when(g + 2 < nch)
                def _():
                    gather_start(g + 2, (b + 2) % _NBUF)

            return carry

        lax.fori_loop(0, nch // _NBUF, outer, 0)
        write_wait((nch - 2) % _NBUF)
        write_wait((nch - 1) % _NBUF)

    return k


@jax.jit
def kernel(x, table):
    b0, b1 = x.shape
    B = b0 * b1
    d = table.shape[1]
    idx = x.reshape(B).astype(jnp.int32)
    table_p = jnp.pad(table, ((0, 0), (0, d)))
    out = _build(B, d, B // _NW, _CHUNK)(idx, table_p)
    return out.reshape(b0, b1, 2 * d)[:, :, :d]
